# initial kernel scaffold (unmeasured)
import functools

import jax
import jax.numpy as jnp
from jax import lax
from jax.experimental import pallas as pl
from jax.experimental.pallas import tpu as pltpu

N_DEV = 8
B = 64
D = 2048
H_SHARD = 4096
H_CHUNK = 512


def _layer_body(x_ref, win_ref, wout_ref, out_ref, acc_ref):
    c = pl.program_id(0)

    @pl.when(c == 0)
    def _():
        acc_ref[...] = jnp.zeros_like(acc_ref)

    h = jnp.dot(x_ref[...], win_ref[...], preferred_element_type=jnp.float32)
    h = jnp.maximum(h, 0.0)
    acc_ref[...] += jnp.dot(h, wout_ref[...], preferred_element_type=jnp.float32)

    @pl.when(c == pl.num_programs(0) - 1)
    def _():
        out_ref[...] = acc_ref[...]


def _layer(x, win, wout):
    n_chunks = H_SHARD // H_CHUNK
    return pl.pallas_call(
        _layer_body,
        grid=(n_chunks,),
        in_specs=[
            pl.BlockSpec((B, D), lambda c: (0, 0)),
            pl.BlockSpec((D, H_CHUNK), lambda c: (0, c)),
            pl.BlockSpec((H_CHUNK, D), lambda c: (c, 0)),
        ],
        out_specs=pl.BlockSpec((B, D), lambda c: (0, 0)),
        out_shape=jax.ShapeDtypeStruct((B, D), jnp.float32),
        scratch_shapes=[pltpu.VMEM((B, D), jnp.float32)],
    )(x, win, wout)


def _allreduce_body(scatter, p_ref, out_ref, comm_ref, send_sems, recv_sems):
    my = lax.axis_index("i")
    left = lax.rem(my - 1 + N_DEV, N_DEV)
    right = lax.rem(my + 1, N_DEV)

    barrier_sem = pltpu.get_barrier_semaphore()
    for nbr in (left, right):
        pl.semaphore_signal(
            barrier_sem, inc=1,
            device_id=(nbr,), device_id_type=pl.DeviceIdType.MESH,
        )
    pl.semaphore_wait(barrier_sem, 2)

    comm_ref[0] = p_ref[...]

    for h in range(N_DEV - 1):
        rdma = pltpu.make_async_remote_copy(
            src_ref=comm_ref.at[h],
            dst_ref=comm_ref.at[h + 1],
            send_sem=send_sems.at[h],
            recv_sem=recv_sems.at[h],
            device_id=(right,),
            device_id_type=pl.DeviceIdType.MESH,
        )
        rdma.start()
        rdma.wait()

    acc = comm_ref[0]
    for s in range(1, N_DEV):
        acc = acc + comm_ref[s]

    if scatter:
        rows = B // N_DEV
        out_ref[...] = lax.dynamic_slice(acc, (my * rows, 0), (rows, D))
    else:
        out_ref[...] = acc


def _allreduce(p, *, collective_id, scatter=False):
    out_rows = B // N_DEV if scatter else B
    return pl.pallas_call(
        functools.partial(_allreduce_body, scatter),
        out_shape=jax.ShapeDtypeStruct((out_rows, D), jnp.float32),
        in_specs=[pl.BlockSpec(memory_space=pltpu.VMEM)],
        out_specs=pl.BlockSpec(memory_space=pltpu.VMEM),
        scratch_shapes=[
            pltpu.VMEM((N_DEV, B, D), jnp.float32),
            pltpu.SemaphoreType.DMA((N_DEV - 1,)),
            pltpu.SemaphoreType.DMA((N_DEV - 1,)),
        ],
        compiler_params=pltpu.CompilerParams(collective_id=collective_id),
    )(p)


def kernel(x, Win0, Wout0, Win1, Wout1, Win2, Wout2):
    p0 = _layer(x, Win0, Wout0)
    x1 = _allreduce(p0, collective_id=0)
    p1 = _layer(x1, Win1, Wout1)
    x2 = _allreduce(p1, collective_id=1)
    p2 = _layer(x2, Win2, Wout2)
    return _allreduce(p2, collective_id=2, scatter=True)


# baseline (device time: 230527 ns/iter reference)
import functools

import jax
import jax.numpy as jnp
from jax import lax
from jax.experimental import pallas as pl
from jax.experimental.pallas import tpu as pltpu

N_DEV = 8
B = 64
D = 2048
H_SHARD = 4096
H_CHUNK = 512


def _layer_body(x_ref, win_ref, wout_ref, out_ref, acc_ref):
    c = pl.program_id(0)

    @pl.when(c == 0)
    def _():
        acc_ref[...] = jnp.zeros_like(acc_ref)

    h = jnp.dot(x_ref[...], win_ref[...], preferred_element_type=jnp.float32)
    h = jnp.maximum(h, 0.0)
    acc_ref[...] += jnp.dot(h, wout_ref[...], preferred_element_type=jnp.float32)

    @pl.when(c == pl.num_programs(0) - 1)
    def _():
        out_ref[...] = acc_ref[...]


def _layer(x, win, wout):
    n_chunks = H_SHARD // H_CHUNK
    return pl.pallas_call(
        _layer_body,
        grid=(n_chunks,),
        in_specs=[
            pl.BlockSpec((B, D), lambda c: (0, 0)),
            pl.BlockSpec((D, H_CHUNK), lambda c: (0, c)),
            pl.BlockSpec((H_CHUNK, D), lambda c: (c, 0)),
        ],
        out_specs=pl.BlockSpec((B, D), lambda c: (0, 0)),
        out_shape=jax.ShapeDtypeStruct((B, D), jnp.float32),
        scratch_shapes=[pltpu.VMEM((B, D), jnp.float32)],
    )(x, win, wout)


def _allreduce_body(scatter, p_ref, out_ref, comm_ref, send_sems, recv_sems):
    my = lax.axis_index("i")
    left = lax.rem(my - 1 + N_DEV, N_DEV)
    right = lax.rem(my + 1, N_DEV)

    barrier_sem = pltpu.get_barrier_semaphore()
    for nbr in (left, right):
        pl.semaphore_signal(
            barrier_sem, inc=1,
            device_id=(nbr,), device_id_type=pl.DeviceIdType.MESH,
        )
    pl.semaphore_wait(barrier_sem, 2)

    comm_ref[0] = p_ref[...]

    for h in range(N_DEV - 1):
        rdma = pltpu.make_async_remote_copy(
            src_ref=comm_ref.at[h],
            dst_ref=comm_ref.at[h + 1],
            send_sem=send_sems.at[h],
            recv_sem=recv_sems.at[h],
            device_id=(right,),
            device_id_type=pl.DeviceIdType.MESH,
        )
        rdma.start()
        rdma.wait()

    if scatter:
        rows = B // N_DEV
        idx = my * rows
        acc = comm_ref[0, pl.ds(idx, rows), :]
        for s in range(1, N_DEV):
            acc = acc + comm_ref[s, pl.ds(idx, rows), :]
        out_ref[...] = acc
    else:
        acc = comm_ref[0]
        for s in range(1, N_DEV):
            acc = acc + comm_ref[s]
        out_ref[...] = acc


def _allreduce(p, *, collective_id, scatter=False):
    out_rows = B // N_DEV if scatter else B
    return pl.pallas_call(
        functools.partial(_allreduce_body, scatter),
        out_shape=jax.ShapeDtypeStruct((out_rows, D), jnp.float32),
        in_specs=[pl.BlockSpec(memory_space=pltpu.VMEM)],
        out_specs=pl.BlockSpec(memory_space=pltpu.VMEM),
        scratch_shapes=[
            pltpu.VMEM((N_DEV, B, D), jnp.float32),
            pltpu.SemaphoreType.DMA((N_DEV - 1,)),
            pltpu.SemaphoreType.DMA((N_DEV - 1,)),
        ],
        compiler_params=pltpu.CompilerParams(collective_id=collective_id),
    )(p)


def kernel(x, Win0, Wout0, Win1, Wout1, Win2, Wout2):
    p0 = _layer(x, Win0, Wout0)
    x1 = _allreduce(p0, collective_id=0)
    p1 = _layer(x1, Win1, Wout1)
    x2 = _allreduce(p1, collective_id=1)
    p2 = _layer(x2, Win2, Wout2)
    return _allreduce(p2, collective_id=2, scatter=True)


# device time: 127064 ns/iter; 1.8143x vs baseline; 1.8143x over previous
import functools

import jax
import jax.numpy as jnp
from jax import lax
from jax.experimental import pallas as pl
from jax.experimental.pallas import tpu as pltpu

N_DEV = 8
B = 64
D = 2048
H_SHARD = 4096
H_CHUNK = 512


def _layer_body(x_ref, win_ref, wout_ref, out_ref, acc_ref):
    c = pl.program_id(0)

    @pl.when(c == 0)
    def _():
        acc_ref[...] = jnp.zeros_like(acc_ref)

    h = jnp.dot(x_ref[...], win_ref[...], preferred_element_type=jnp.float32)
    h = jnp.maximum(h, 0.0)
    acc_ref[...] += jnp.dot(h, wout_ref[...], preferred_element_type=jnp.float32)

    @pl.when(c == pl.num_programs(0) - 1)
    def _():
        out_ref[...] = acc_ref[...]


def _layer(x, win, wout):
    n_chunks = H_SHARD // H_CHUNK
    return pl.pallas_call(
        _layer_body,
        grid=(n_chunks,),
        in_specs=[
            pl.BlockSpec((B, D), lambda c: (0, 0)),
            pl.BlockSpec((D, H_CHUNK), lambda c: (0, c)),
            pl.BlockSpec((H_CHUNK, D), lambda c: (c, 0)),
        ],
        out_specs=pl.BlockSpec((B, D), lambda c: (0, 0)),
        out_shape=jax.ShapeDtypeStruct((B, D), jnp.float32),
        scratch_shapes=[pltpu.VMEM((B, D), jnp.float32)],
    )(x, win, wout)


_R = B // N_DEV


def _allreduce_body(scatter, p_ref, out_ref, comm_ref, rs_ref, send_sems, recv_sems):
    my = lax.axis_index("i")

    barrier_sem = pltpu.get_barrier_semaphore()
    for d in (1, 2, 4):
        pl.semaphore_signal(
            barrier_sem, inc=1,
            device_id=(jnp.bitwise_xor(my, d),),
            device_id_type=pl.DeviceIdType.MESH,
        )
    pl.semaphore_wait(barrier_sem, 3)

    comm_ref[...] = p_ref[...]

    stage_off = 0
    for r, d in enumerate((4, 2, 1)):
        n = d * _R
        keep_c = jnp.bitwise_and(my, 7 & ~(d - 1))
        send_c = jnp.bitwise_xor(keep_c, d)
        partner = jnp.bitwise_xor(my, d)
        rdma = pltpu.make_async_remote_copy(
            src_ref=comm_ref.at[pl.ds(send_c * _R, n)],
            dst_ref=rs_ref.at[pl.ds(stage_off, n)],
            send_sem=send_sems.at[r],
            recv_sem=recv_sems.at[r],
            device_id=(partner,),
            device_id_type=pl.DeviceIdType.MESH,
        )
        rdma.start()
        rdma.wait()
        ks = keep_c * _R
        comm_ref[pl.ds(ks, n), :] = (
            comm_ref[pl.ds(ks, n), :] + rs_ref[pl.ds(stage_off, n), :]
        )
        stage_off += n

    if scatter:
        out_ref[...] = comm_ref[pl.ds(my * _R, _R), :]
        return

    for j, d in enumerate((1, 2, 4)):
        r = 3 + j
        n = d * _R
        own_c = jnp.bitwise_and(my, 7 & ~(d - 1))
        partner = jnp.bitwise_xor(my, d)
        rdma = pltpu.make_async_remote_copy(
            src_ref=comm_ref.at[pl.ds(own_c * _R, n)],
            dst_ref=comm_ref.at[pl.ds(own_c * _R, n)],
            send_sem=send_sems.at[r],
            recv_sem=recv_sems.at[r],
            device_id=(partner,),
            device_id_type=pl.DeviceIdType.MESH,
        )
        rdma.start()
        rdma.wait()

    out_ref[...] = comm_ref[...]


def _allreduce(p, *, collective_id, scatter=False):
    out_rows = B // N_DEV if scatter else B
    return pl.pallas_call(
        functools.partial(_allreduce_body, scatter),
        out_shape=jax.ShapeDtypeStruct((out_rows, D), jnp.float32),
        in_specs=[pl.BlockSpec(memory_space=pltpu.VMEM)],
        out_specs=pl.BlockSpec(memory_space=pltpu.VMEM),
        scratch_shapes=[
            pltpu.VMEM((B, D), jnp.float32),
            pltpu.VMEM((B - _R, D), jnp.float32),
            pltpu.SemaphoreType.DMA((6,)),
            pltpu.SemaphoreType.DMA((6,)),
        ],
        compiler_params=pltpu.CompilerParams(collective_id=collective_id),
    )(p)


def kernel(x, Win0, Wout0, Win1, Wout1, Win2, Wout2):
    p0 = _layer(x, Win0, Wout0)
    x1 = _allreduce(p0, collective_id=0)
    p1 = _layer(x1, Win1, Wout1)
    x2 = _allreduce(p1, collective_id=1)
    p2 = _layer(x2, Win2, Wout2)
    return _allreduce(p2, collective_id=2, scatter=True)


# device time: 124332 ns/iter; 1.8541x vs baseline; 1.0220x over previous
import functools

import jax
import jax.numpy as jnp
from jax import lax
from jax.experimental import pallas as pl
from jax.experimental.pallas import tpu as pltpu

N_DEV = 8
B = 64
D = 2048
H_SHARD = 4096
H_CHUNK = 512


def _layer_body(x_ref, win_ref, wout_ref, out_ref, acc_ref):
    c = pl.program_id(0)

    @pl.when(c == 0)
    def _():
        acc_ref[...] = jnp.zeros_like(acc_ref)

    h = jnp.dot(x_ref[...], win_ref[...], preferred_element_type=jnp.float32)
    h = jnp.maximum(h, 0.0)
    acc_ref[...] += jnp.dot(h, wout_ref[...], preferred_element_type=jnp.float32)

    @pl.when(c == pl.num_programs(0) - 1)
    def _():
        out_ref[...] = acc_ref[...]


def _layer(x, win, wout):
    n_chunks = H_SHARD // H_CHUNK
    return pl.pallas_call(
        _layer_body,
        grid=(n_chunks,),
        in_specs=[
            pl.BlockSpec((B, D), lambda c: (0, 0)),
            pl.BlockSpec((D, H_CHUNK), lambda c: (0, c)),
            pl.BlockSpec((H_CHUNK, D), lambda c: (c, 0)),
        ],
        out_specs=pl.BlockSpec((B, D), lambda c: (0, 0)),
        out_shape=jax.ShapeDtypeStruct((B, D), jnp.float32),
        scratch_shapes=[pltpu.VMEM((B, D), jnp.float32)],
    )(x, win, wout)


_R = B // N_DEV


def _allreduce_body(scatter, p_ref, out_ref, comm_ref, rs_ref, send_sems, recv_sems):
    my = lax.axis_index("i")

    barrier_sem = pltpu.get_barrier_semaphore()
    for g in (1, 3, 4):
        pl.semaphore_signal(
            barrier_sem, inc=1,
            device_id=(jnp.bitwise_xor(my, g),),
            device_id_type=pl.DeviceIdType.MESH,
        )
    pl.semaphore_wait(barrier_sem, 3)

    comm_ref[...] = p_ref[...]

    stage_off = 0
    for r, (g, d) in enumerate(((4, 4), (3, 2), (1, 1))):
        n = d * _R
        keep_c = jnp.bitwise_and(my, 7 & ~(d - 1))
        send_c = jnp.bitwise_xor(keep_c, g)
        send_c = jnp.bitwise_and(send_c, 7 & ~(d - 1))
        partner = jnp.bitwise_xor(my, g)
        rdma = pltpu.make_async_remote_copy(
            src_ref=comm_ref.at[pl.ds(send_c * _R, n)],
            dst_ref=rs_ref.at[pl.ds(stage_off, n)],
            send_sem=send_sems.at[r],
            recv_sem=recv_sems.at[r],
            device_id=(partner,),
            device_id_type=pl.DeviceIdType.MESH,
        )
        rdma.start()
        rdma.wait()
        ks = keep_c * _R
        comm_ref[pl.ds(ks, n), :] = (
            comm_ref[pl.ds(ks, n), :] + rs_ref[pl.ds(stage_off, n), :]
        )
        stage_off += n

    if scatter:
        out_ref[...] = comm_ref[pl.ds(my * _R, _R), :]
        return

    for j, (g, d) in enumerate(((1, 1), (3, 2), (4, 4))):
        r = 3 + j
        n = d * _R
        own_c = jnp.bitwise_and(my, 7 & ~(d - 1))
        partner = jnp.bitwise_xor(my, g)
        rdma = pltpu.make_async_remote_copy(
            src_ref=comm_ref.at[pl.ds(own_c * _R, n)],
            dst_ref=comm_ref.at[pl.ds(own_c * _R, n)],
            send_sem=send_sems.at[r],
            recv_sem=recv_sems.at[r],
            device_id=(partner,),
            device_id_type=pl.DeviceIdType.MESH,
        )
        rdma.start()
        rdma.wait()

    out_ref[...] = comm_ref[...]


def _allreduce(p, *, collective_id, scatter=False):
    out_rows = B // N_DEV if scatter else B
    return pl.pallas_call(
        functools.partial(_allreduce_body, scatter),
        out_shape=jax.ShapeDtypeStruct((out_rows, D), jnp.float32),
        in_specs=[pl.BlockSpec(memory_space=pltpu.VMEM)],
        out_specs=pl.BlockSpec(memory_space=pltpu.VMEM),
        scratch_shapes=[
            pltpu.VMEM((B, D), jnp.float32),
            pltpu.VMEM((B - _R, D), jnp.float32),
            pltpu.SemaphoreType.DMA((6,)),
            pltpu.SemaphoreType.DMA((6,)),
        ],
        compiler_params=pltpu.CompilerParams(collective_id=collective_id),
    )(p)


def kernel(x, Win0, Wout0, Win1, Wout1, Win2, Wout2):
    p0 = _layer(x, Win0, Wout0)
    x1 = _allreduce(p0, collective_id=0)
    p1 = _layer(x1, Win1, Wout1)
    x2 = _allreduce(p1, collective_id=1)
    p2 = _layer(x2, Win2, Wout2)
    return _allreduce(p2, collective_id=2, scatter=True)


# device time: 111514 ns/iter; 2.0672x vs baseline; 1.1149x over previous
import functools

import jax
import jax.numpy as jnp
from jax import lax
from jax.experimental import pallas as pl
from jax.experimental.pallas import tpu as pltpu

N_DEV = 8
B = 64
D = 2048
H_SHARD = 4096
H_CHUNK = 512


def _layer_body(x_ref, win_ref, wout_ref, out_ref, acc_ref):
    c = pl.program_id(0)

    @pl.when(c == 0)
    def _():
        acc_ref[...] = jnp.zeros_like(acc_ref)

    h = jnp.dot(x_ref[...], win_ref[...], preferred_element_type=jnp.float32)
    h = jnp.maximum(h, 0.0)
    acc_ref[...] += jnp.dot(h, wout_ref[...], preferred_element_type=jnp.float32)

    @pl.when(c == pl.num_programs(0) - 1)
    def _():
        out_ref[...] = acc_ref[...]


def _layer(x, win, wout):
    n_chunks = H_SHARD // H_CHUNK
    return pl.pallas_call(
        _layer_body,
        grid=(n_chunks,),
        in_specs=[
            pl.BlockSpec((B, D), lambda c: (0, 0)),
            pl.BlockSpec((D, H_CHUNK), lambda c: (0, c)),
            pl.BlockSpec((H_CHUNK, D), lambda c: (c, 0)),
        ],
        out_specs=pl.BlockSpec((B, D), lambda c: (0, 0)),
        out_shape=jax.ShapeDtypeStruct((B, D), jnp.float32),
        scratch_shapes=[pltpu.VMEM((B, D), jnp.float32)],
    )(x, win, wout)


_R = B // N_DEV


def _allreduce_body(scatter, p_ref, out_ref, comm_ref, xb_ref, rs_ref, send_sems, recv_sems):
    my = lax.axis_index("i")

    barrier_sem = pltpu.get_barrier_semaphore()
    for g in (1, 3, 4):
        pl.semaphore_signal(
            barrier_sem, inc=1,
            device_id=(jnp.bitwise_xor(my, g),),
            device_id_type=pl.DeviceIdType.MESH,
        )
    pl.semaphore_wait(barrier_sem, 3)

    comm_ref[...] = p_ref[...]
    xb_ref[...] = p_ref[...].astype(jnp.bfloat16)

    stage_off = 0
    for r, (g, d) in enumerate(((4, 4), (3, 2), (1, 1))):
        n = d * _R
        keep_c = jnp.bitwise_and(my, 7 & ~(d - 1))
        send_c = jnp.bitwise_xor(keep_c, g)
        send_c = jnp.bitwise_and(send_c, 7 & ~(d - 1))
        partner = jnp.bitwise_xor(my, g)
        rdma = pltpu.make_async_remote_copy(
            src_ref=xb_ref.at[pl.ds(send_c * _R, n)],
            dst_ref=rs_ref.at[pl.ds(stage_off, n)],
            send_sem=send_sems.at[r],
            recv_sem=recv_sems.at[r],
            device_id=(partner,),
            device_id_type=pl.DeviceIdType.MESH,
        )
        rdma.start()
        rdma.wait()
        ks = keep_c * _R
        acc = comm_ref[pl.ds(ks, n), :] + rs_ref[pl.ds(stage_off, n), :].astype(
            jnp.float32
        )
        comm_ref[pl.ds(ks, n), :] = acc
        xb_ref[pl.ds(ks, n), :] = acc.astype(jnp.bfloat16)
        stage_off += n

    if scatter:
        out_ref[...] = comm_ref[pl.ds(my * _R, _R), :]
        return

    for j, (g, d) in enumerate(((1, 1), (3, 2), (4, 4))):
        r = 3 + j
        n = d * _R
        own_c = jnp.bitwise_and(my, 7 & ~(d - 1))
        partner = jnp.bitwise_xor(my, g)
        rdma = pltpu.make_async_remote_copy(
            src_ref=xb_ref.at[pl.ds(own_c * _R, n)],
            dst_ref=xb_ref.at[pl.ds(own_c * _R, n)],
            send_sem=send_sems.at[r],
            recv_sem=recv_sems.at[r],
            device_id=(partner,),
            device_id_type=pl.DeviceIdType.MESH,
        )
        rdma.start()
        rdma.wait()
        rcv_c = jnp.bitwise_and(partner, 7 & ~(d - 1))
        comm_ref[pl.ds(rcv_c * _R, n), :] = xb_ref[
            pl.ds(rcv_c * _R, n), :
        ].astype(jnp.float32)

    out_ref[...] = comm_ref[...]


def _allreduce(p, *, collective_id, scatter=False):
    out_rows = B // N_DEV if scatter else B
    return pl.pallas_call(
        functools.partial(_allreduce_body, scatter),
        out_shape=jax.ShapeDtypeStruct((out_rows, D), jnp.float32),
        in_specs=[pl.BlockSpec(memory_space=pltpu.VMEM)],
        out_specs=pl.BlockSpec(memory_space=pltpu.VMEM),
        scratch_shapes=[
            pltpu.VMEM((B, D), jnp.float32),
            pltpu.VMEM((B, D), jnp.bfloat16),
            pltpu.VMEM((B - _R, D), jnp.bfloat16),
            pltpu.SemaphoreType.DMA((6,)),
            pltpu.SemaphoreType.DMA((6,)),
        ],
        compiler_params=pltpu.CompilerParams(collective_id=collective_id),
    )(p)


def kernel(x, Win0, Wout0, Win1, Wout1, Win2, Wout2):
    p0 = _layer(x, Win0, Wout0)
    x1 = _allreduce(p0, collective_id=0)
    p1 = _layer(x1, Win1, Wout1)
    x2 = _allreduce(p1, collective_id=1)
    p2 = _layer(x2, Win2, Wout2)
    return _allreduce(p2, collective_id=2, scatter=True)


# device time: 97656 ns/iter; 2.3606x vs baseline; 1.1419x over previous
import functools

import jax
import jax.numpy as jnp
from jax import lax
from jax.experimental import pallas as pl
from jax.experimental.pallas import tpu as pltpu

N_DEV = 8
B = 64
D = 2048
H_SHARD = 4096
H_CHUNK = 512


def _layer_body(x_ref, win_ref, wout_ref, out_ref, acc_ref):
    c = pl.program_id(0)

    @pl.when(c == 0)
    def _():
        acc_ref[...] = jnp.zeros_like(acc_ref)

    h = jnp.dot(x_ref[...], win_ref[...], preferred_element_type=jnp.float32)
    h = jnp.maximum(h, 0.0)
    acc_ref[...] += jnp.dot(h, wout_ref[...], preferred_element_type=jnp.float32)

    @pl.when(c == pl.num_programs(0) - 1)
    def _():
        out_ref[...] = acc_ref[...]


def _layer(x, win, wout):
    n_chunks = H_SHARD // H_CHUNK
    return pl.pallas_call(
        _layer_body,
        grid=(n_chunks,),
        in_specs=[
            pl.BlockSpec((B, D), lambda c: (0, 0)),
            pl.BlockSpec((D, H_CHUNK), lambda c: (0, c)),
            pl.BlockSpec((H_CHUNK, D), lambda c: (c, 0)),
        ],
        out_specs=pl.BlockSpec((B, D), lambda c: (0, 0)),
        out_shape=jax.ShapeDtypeStruct((B, D), jnp.float32),
        scratch_shapes=[pltpu.VMEM((B, D), jnp.float32)],
    )(x, win, wout)


_R = B // N_DEV


def _allreduce_body(scatter, p_ref, out_ref, xb_ref, rs_ref, send_sems, recv_sems):
    my = lax.axis_index("i")

    barrier_sem = pltpu.get_barrier_semaphore()
    for r in range(1, N_DEV):
        pl.semaphore_signal(
            barrier_sem, inc=1,
            device_id=(jnp.bitwise_xor(my, r),),
            device_id_type=pl.DeviceIdType.MESH,
        )
    pl.semaphore_wait(barrier_sem, N_DEV - 1)

    xb_ref[...] = p_ref[...].astype(jnp.bfloat16)

    rs_rdmas = []
    for r in range(1, N_DEV):
        partner = jnp.bitwise_xor(my, r)
        rdma = pltpu.make_async_remote_copy(
            src_ref=xb_ref.at[pl.ds(partner * _R, _R)],
            dst_ref=rs_ref.at[r - 1],
            send_sem=send_sems.at[r - 1],
            recv_sem=recv_sems.at[r - 1],
            device_id=(partner,),
            device_id_type=pl.DeviceIdType.MESH,
        )
        rdma.start()
        rs_rdmas.append(rdma)
    for rdma in rs_rdmas:
        rdma.wait()

    acc = p_ref[pl.ds(my * _R, _R), :]
    for r in range(1, N_DEV):
        acc = acc + rs_ref[r - 1].astype(jnp.float32)

    if scatter:
        out_ref[...] = acc
        return

    xb_ref[pl.ds(my * _R, _R), :] = acc.astype(jnp.bfloat16)
    ag_rdmas = []
    for r in range(1, N_DEV):
        partner = jnp.bitwise_xor(my, r)
        rdma = pltpu.make_async_remote_copy(
            src_ref=xb_ref.at[pl.ds(my * _R, _R)],
            dst_ref=xb_ref.at[pl.ds(my * _R, _R)],
            send_sem=send_sems.at[N_DEV - 1 + r - 1],
            recv_sem=recv_sems.at[N_DEV - 1 + r - 1],
            device_id=(partner,),
            device_id_type=pl.DeviceIdType.MESH,
        )
        rdma.start()
        ag_rdmas.append(rdma)
    for rdma in ag_rdmas:
        rdma.wait()

    out_ref[...] = xb_ref[...].astype(jnp.float32)
    out_ref[pl.ds(my * _R, _R), :] = acc


def _allreduce(p, *, collective_id, scatter=False):
    out_rows = B // N_DEV if scatter else B
    return pl.pallas_call(
        functools.partial(_allreduce_body, scatter),
        out_shape=jax.ShapeDtypeStruct((out_rows, D), jnp.float32),
        in_specs=[pl.BlockSpec(memory_space=pltpu.VMEM)],
        out_specs=pl.BlockSpec(memory_space=pltpu.VMEM),
        scratch_shapes=[
            pltpu.VMEM((B, D), jnp.bfloat16),
            pltpu.VMEM((N_DEV - 1, _R, D), jnp.bfloat16),
            pltpu.SemaphoreType.DMA((2 * (N_DEV - 1),)),
            pltpu.SemaphoreType.DMA((2 * (N_DEV - 1),)),
        ],
        compiler_params=pltpu.CompilerParams(collective_id=collective_id),
    )(p)


def kernel(x, Win0, Wout0, Win1, Wout1, Win2, Wout2):
    p0 = _layer(x, Win0, Wout0)
    x1 = _allreduce(p0, collective_id=0)
    p1 = _layer(x1, Win1, Wout1)
    x2 = _allreduce(p1, collective_id=1)
    p2 = _layer(x2, Win2, Wout2)
    return _allreduce(p2, collective_id=2, scatter=True)
